# Initial kernel scaffold; baseline (speedup 1.0000x reference)
import jax
import jax.numpy as jnp
from jax.experimental import pallas as pl


def kernel(x, Wg, Wdw, bdw, Wup, bup):
    raise NotImplementedError("write your pallas kernel here")


# speedup vs baseline: 3.5115x; 3.5115x over previous
import functools

import jax
import jax.numpy as jnp
import numpy as np
from jax.experimental import pallas as pl
from jax.experimental.pallas import tpu as pltpu


def _moe_body(x_ref, wg_ref, wdw_ref, bdw_ref, wup_ref, bup_ref,
              out_ref, lb_ref, eidx_scr, h_scr):
    e = pl.program_id(0)
    n_e = pl.num_programs(0)
    N = x_ref.shape[0]

    @pl.when(e == 0)
    def _gate():
        logits = jax.lax.dot_general(
            x_ref[...], wg_ref[...], (((1,), (0,)), ((), ())),
            preferred_element_type=jnp.float32)
        m = jnp.max(logits, axis=1, keepdims=True)
        iota_e = jax.lax.broadcasted_iota(jnp.int32, logits.shape, 1)
        idx = jnp.min(jnp.where(logits == m, iota_e, logits.shape[1]),
                      axis=1)
        eidx_scr[...] = idx[:, None]
        onehot = (iota_e == idx[:, None]).astype(jnp.float32)
        counts = jnp.sum(onehot, axis=0)
        frac = counts * (1.0 / N)
        lb = logits.shape[1] * jnp.sum(frac * frac) * 0.1
        lb_ref[...] = jnp.broadcast_to(lb, (1, 1))
        h_scr[...] = jnp.zeros_like(h_scr)

    w = wdw_ref[0]
    he = jax.lax.dot_general(
        x_ref[...], w, (((1,), (1,)), ((), ())),
        preferred_element_type=jnp.float32)
    mask = eidx_scr[...] == e
    he = he + bdw_ref[0, 0][None, :]
    h_scr[...] += jnp.where(mask, he, 0.0)

    @pl.when(e == n_e - 1)
    def _up():
        h = h_scr[...]
        act = 0.5 * h * (1.0 + jnp.tanh(
            np.sqrt(2.0 / np.pi) * (h + 0.044715 * h * h * h)))
        out_ref[...] = jax.lax.dot_general(
            act, wup_ref[...], (((1,), (1,)), ((), ())),
            preferred_element_type=jnp.float32) + bup_ref[...][None, :]


def kernel(x, Wg, Wdw, bdw, Wup, bup):
    B, S, D = x.shape
    E, DOWN, _ = Wdw.shape
    N = B * S
    xf = x.reshape(N, D)

    out, lb = pl.pallas_call(
        _moe_body,
        grid=(E,),
        in_specs=[
            pl.BlockSpec((N, D), lambda e: (0, 0)),
            pl.BlockSpec((D, E), lambda e: (0, 0)),
            pl.BlockSpec((1, DOWN, D), lambda e: (e, 0, 0)),
            pl.BlockSpec((1, 1, DOWN), lambda e: (e, 0, 0)),
            pl.BlockSpec((D, DOWN), lambda e: (0, 0)),
            pl.BlockSpec((D,), lambda e: (0,)),
        ],
        out_specs=[
            pl.BlockSpec((N, D), lambda e: (0, 0)),
            pl.BlockSpec((1, 1), lambda e: (0, 0)),
        ],
        out_shape=[
            jax.ShapeDtypeStruct((N, D), jnp.float32),
            jax.ShapeDtypeStruct((1, 1), jnp.float32),
        ],
        scratch_shapes=[
            pltpu.VMEM((N, 1), jnp.int32),
            pltpu.VMEM((N, DOWN), jnp.float32),
        ],
        compiler_params=pltpu.CompilerParams(
            dimension_semantics=("arbitrary",)),
    )(xf, Wg, Wdw, bdw.reshape(E, 1, DOWN), Wup, bup)

    return out.reshape(B, S, D), lb.reshape(())
